# baseline (speedup 1.0000x reference)
import functools

import jax
import jax.numpy as jnp
from jax import lax
from jax.experimental import pallas as pl
from jax.experimental.pallas import tpu as pltpu
from jax.experimental.pallas import tpu_sc as plsc

HIDDEN = 64
BATCH = 16384
HIST = 50
TOTAL = BATCH * HIST

NUM_CORES = 2
NUM_SUBCORES = 16
NUM_WORKERS = NUM_CORES * NUM_SUBCORES
PER_WORKER = TOTAL // NUM_WORKERS
CHUNK = 256
NCHUNK = PER_WORKER // CHUNK
NBUF = 4
NROUND = NCHUNK // NBUF

_mesh = plsc.VectorSubcoreMesh(core_axis_name="c", subcore_axis_name="s")


@functools.partial(
    pl.kernel,
    out_type=jax.ShapeDtypeStruct((TOTAL, HIDDEN), jnp.float32),
    mesh=_mesh,
    scratch_types=[
        pltpu.VMEM((NCHUNK, CHUNK), jnp.int32),
        [pltpu.VMEM((CHUNK, HIDDEN), jnp.float32) for _ in range(NBUF)],
        [pltpu.SemaphoreType.DMA for _ in range(NBUF)],
        [pltpu.SemaphoreType.DMA for _ in range(NBUF)],
    ],
    compiler_params=pltpu.CompilerParams(use_tc_tiling_on_sc=False),
)
def _gather_kernel(idx_hbm, table_hbm, out_hbm, idx_v, rows, g_sem, w_sem):
    wid = lax.axis_index("s") * NUM_CORES + lax.axis_index("c")
    base = wid * NCHUNK

    def out_slice(i):
        return out_hbm.at[pl.ds((base + i) * CHUNK, CHUNK)]

    pltpu.sync_copy(idx_hbm.at[pl.ds(base, NCHUNK)], idx_v)

    for b in range(NBUF):
        pltpu.async_copy(table_hbm.at[idx_v.at[b]], rows[b], g_sem[b])

    def round_body(r, carry):
        g = r * NBUF
        for b in range(NBUF):
            pltpu.make_async_copy(table_hbm.at[idx_v.at[g + b]], rows[b],
                                  g_sem[b]).wait()
            pltpu.async_copy(rows[b], out_slice(g + b), w_sem[b])
        for b in range(NBUF):
            pltpu.make_async_copy(rows[b], out_slice(g + b), w_sem[b]).wait()
            pltpu.async_copy(table_hbm.at[idx_v.at[g + NBUF + b]], rows[b],
                             g_sem[b])
        return carry

    lax.fori_loop(0, NROUND - 1, round_body, 0)

    g = (NROUND - 1) * NBUF
    for b in range(NBUF):
        pltpu.make_async_copy(table_hbm.at[idx_v.at[g + b]], rows[b],
                              g_sem[b]).wait()
        pltpu.async_copy(rows[b], out_slice(g + b), w_sem[b])
    for b in range(NBUF):
        pltpu.make_async_copy(rows[b], out_slice(g + b), w_sem[b]).wait()


def kernel(inputs, weight):
    idx = inputs.reshape(TOTAL // CHUNK, CHUNK).astype(jnp.int32)
    out = _gather_kernel(idx, weight)
    return out.reshape(inputs.shape + (HIDDEN,))
